# baseline (speedup 1.0000x reference)
import functools
import math

import jax
import jax.numpy as jnp
from jax import lax
from jax.experimental import pallas as pl
from jax.experimental.pallas import tpu as pltpu
from jax.experimental.pallas import tpu_sc as plsc

_N = 32768
_B = 16
_D = 128
_DP = 64
_H = 4
_BN = 4096
_G = _N // _BN
_SCALE = 1.0 / math.sqrt(float(_DP))

_NW = 32
_TPW = _N // _NW


def _tc_body(x_ref, seg_ref, wk_ref, wqbd_ref, e_ref, stats_ref, u_ref):
    g = pl.program_id(0)

    @pl.when(g == 0)
    def _fold_u():
        u_ref[...] = lax.dot_general(wk_ref[...], wqbd_ref[...],
                                     (((1,), (0,)), ((), ())),
                                     precision=lax.Precision.HIGHEST)

    t = lax.dot_general(x_ref[...], u_ref[...], (((1,), (0,)), ((), ())))
    t = t * _SCALE
    e0 = jnp.exp(jnp.minimum(t, 50.0))
    e_ref[...] = jnp.transpose(e0)
    ohT = (lax.broadcasted_iota(jnp.int32, (_B, _BN), 0)
           == seg_ref[...]).astype(jnp.float32)
    part = jnp.transpose(
        lax.dot_general(ohT, e0, (((1,), (0,)), ((), ()))))

    @pl.when(g == 0)
    def _init():
        stats_ref[...] = part

    @pl.when(g != 0)
    def _acc():
        stats_ref[...] = stats_ref[...] + part


def _make_tc_call(interpret=False):
    return pl.pallas_call(
        _tc_body,
        grid=(_G,),
        in_specs=[
            pl.BlockSpec((_BN, _D), lambda g: (g, 0)),
            pl.BlockSpec((1, _BN), lambda g: (0, g)),
            pl.BlockSpec((_D, _H * _DP), lambda g: (0, 0)),
            pl.BlockSpec((_H * _DP, _H), lambda g: (0, 0)),
        ],
        out_specs=[
            pl.BlockSpec((_H, _BN), lambda g: (0, g)),
            pl.BlockSpec((_H, _B), lambda g: (0, 0)),
        ],
        out_shape=[
            jax.ShapeDtypeStruct((_H, _N), jnp.float32),
            jax.ShapeDtypeStruct((_H, _B), jnp.float32),
        ],
        scratch_shapes=[
            pltpu.VMEM((_D, _H), jnp.float32),
        ],
        interpret=interpret,
    )


@functools.partial(
    pl.kernel,
    mesh=plsc.VectorSubcoreMesh(core_axis_name="c", subcore_axis_name="s"),
    out_type=jax.ShapeDtypeStruct((_H * _N,), jnp.float32),
    scratch_types=[
        pltpu.VMEM((_H * _TPW,), jnp.float32),
        pltpu.VMEM((_TPW,), jnp.int32),
        pltpu.VMEM((_H * _B,), jnp.float32),
        pltpu.VMEM((_H * _TPW,), jnp.float32),
    ],
)
def _sc_normalize(e_hbm, seg_hbm, stats_hbm, out_hbm,
                  e_v, seg_v, stats_v, out_v):
    wid = lax.axis_index("s") * 2 + lax.axis_index("c")
    tok0 = wid * _TPW
    pltpu.sync_copy(seg_hbm.at[pl.ds(tok0, _TPW)], seg_v)
    for h in range(_H):
        pltpu.sync_copy(e_hbm.at[pl.ds(h * _N + tok0, _TPW)],
                        e_v.at[pl.ds(h * _TPW, _TPW)])
    pltpu.sync_copy(stats_hbm, stats_v)
    recip = [1.0 / jnp.maximum(stats_v[pl.ds(h * _B, _B)], 1e-30)
             for h in range(_H)]
    dnums = lax.GatherDimensionNumbers(offset_dims=(), collapsed_slice_dims=(0,),
                                       start_index_map=(0,))
    inb = lax.GatherScatterMode.PROMISE_IN_BOUNDS
    for j in range(_TPW // 16):
        seg_vec = seg_v[pl.ds(j * 16, 16)]
        for h in range(_H):
            rc = lax.gather(recip[h], seg_vec[:, None], dnums,
                            slice_sizes=(1,), mode=inb)
            base = h * _TPW + j * 16
            out_v[pl.ds(base, 16)] = e_v[pl.ds(base, 16)] * rc
    for h in range(_H):
        pltpu.sync_copy(out_v.at[pl.ds(h * _TPW, _TPW)],
                        out_hbm.at[pl.ds(h * _N + tok0, _TPW)])


def kernel(inputs, segment_ids, lengths, W1, b1, W2, b2, W3, b3, Wr, br, W_k, W_q):
    seg_i = segment_ids.astype(jnp.int32)
    seg_row = seg_i.reshape(1, _N)
    wk_top = W_k[:_D, :]
    eye = jnp.eye(_H, dtype=jnp.float32)
    wqbd = (W_q[:, :, None] * eye[:, None, :]).reshape(_H * _DP, _H)

    e2d, stats = _make_tc_call()(inputs, seg_row, wk_top, wqbd)
    out_flat = _sc_normalize(e2d.reshape(_H * _N), seg_i, stats.reshape(_H * _B))
    return out_flat.reshape(_H, _N, 1)


# speedup vs baseline: 2.1548x; 2.1548x over previous
import math

import jax
import jax.numpy as jnp
from jax import lax
from jax.experimental import pallas as pl
from jax.experimental.pallas import tpu as pltpu

_N = 32768
_B = 16
_D = 128
_DP = 64
_H = 4
_BN = 16384
_G = _N // _BN
_SCALE = 1.0 / math.sqrt(float(_DP))


def _body(x_ref, seg_ref, wk_ref, wqbd_ref, out_ref, e_ref, stats_ref, u_ref):
    p = pl.program_id(0)
    g = pl.program_id(1)
    seg = jnp.broadcast_to(seg_ref[...], (_H, _BN))

    @pl.when((p == 0) & (g == 0))
    def _fold_u():
        u_ref[...] = lax.dot_general(wk_ref[...], wqbd_ref[...],
                                     (((1,), (0,)), ((), ())),
                                     precision=lax.Precision.HIGHEST) * _SCALE

    @pl.when(p == 0)
    def _phase0():
        t = lax.dot_general(x_ref[...], u_ref[...], (((1,), (0,)), ((), ())))
        e = jnp.exp(jnp.minimum(jnp.transpose(t), 50.0))
        e_ref[:, pl.ds(g * _BN, _BN)] = e
        out_ref[...] = e
        ohT = (lax.broadcasted_iota(jnp.int32, (_B, _BN), 0)
               == seg_ref[...]).astype(jnp.float32)
        part = jnp.transpose(
            lax.dot_general(ohT, e, (((1,), (1,)), ((), ()))))

        @pl.when(g == 0)
        def _init():
            stats_ref[...] = part

        @pl.when(g != 0)
        def _acc():
            stats_ref[...] = stats_ref[...] + part

    @pl.when(p == 1)
    def _phase1():
        e = e_ref[:, pl.ds(g * _BN, _BN)]
        recip = 1.0 / jnp.maximum(stats_ref[...], 1e-30)
        rg = jnp.broadcast_to(recip[:, 0:1], (_H, _BN))
        for s in range(1, _B):
            rg = jnp.where(seg == s, jnp.broadcast_to(recip[:, s:s + 1], (_H, _BN)), rg)
        out_ref[...] = e * rg


def _make_call(interpret=False):
    return pl.pallas_call(
        _body,
        grid=(2, _G),
        in_specs=[
            pl.BlockSpec((_BN, _D), lambda p, g: (g * (1 - p), 0)),
            pl.BlockSpec((1, _BN), lambda p, g: (0, g)),
            pl.BlockSpec((_D, _H * _DP), lambda p, g: (0, 0)),
            pl.BlockSpec((_H * _DP, _H), lambda p, g: (0, 0)),
        ],
        out_specs=pl.BlockSpec((_H, _BN), lambda p, g: (0, g)),
        out_shape=jax.ShapeDtypeStruct((_H, _N), jnp.float32),
        scratch_shapes=[
            pltpu.VMEM((_H, _N), jnp.float32),
            pltpu.VMEM((_H, _B), jnp.float32),
            pltpu.VMEM((_D, _H), jnp.float32),
        ],
        interpret=interpret,
    )


def kernel(inputs, segment_ids, lengths, W1, b1, W2, b2, W3, b3, Wr, br, W_k, W_q):
    seg_row = segment_ids.astype(jnp.int32).reshape(1, _N)
    wk_top = W_k[:_D, :]
    eye = jnp.eye(_H, dtype=jnp.float32)
    wqbd = (W_q[:, :, None] * eye[:, None, :]).reshape(_H * _DP, _H)

    out = _make_call()(inputs, seg_row, wk_top, wqbd)
    return out.reshape(_H, _N, 1)
